# baseline (speedup 1.0000x reference)
import functools

import jax
import jax.numpy as jnp
from jax import lax
from jax.experimental import pallas as pl
from jax.experimental.pallas import tpu as pltpu
from jax.experimental.pallas import tpu_sc as plsc

N = 10000
E = 320000
F_IN = 128
H1 = 128
H2 = 2
CLASSES = 10
NEG_SLOPE = 0.2

NC = 2
NS = 16
NW = NC * NS
EPT = E // NW
K = 80
CH = EPT // K
N_ACC = 10240
STRIPE = N_ACC // NS
ZR = 128
ROW_BLK = 2000
GRID = N // ROW_BLK


def _tc1_body(x_ref, w_ref, att_ref, h_ref, a_ref):
    h = jnp.dot(x_ref[...], w_ref[...], preferred_element_type=jnp.float32)
    h_ref[...] = h
    a_ref[...] = jnp.dot(h, att_ref[...], preferred_element_type=jnp.float32)


_tc1 = pl.pallas_call(
    _tc1_body,
    grid=(GRID,),
    in_specs=[
        pl.BlockSpec((ROW_BLK, F_IN), lambda i: (i, 0)),
        pl.BlockSpec((F_IN, H1), lambda i: (0, 0)),
        pl.BlockSpec((H1, 2), lambda i: (0, 0)),
    ],
    out_specs=[
        pl.BlockSpec((ROW_BLK, H1), lambda i: (i, 0)),
        pl.BlockSpec((ROW_BLK, 2), lambda i: (i, 0)),
    ],
    out_shape=[
        jax.ShapeDtypeStruct((N, H1), jnp.float32),
        jax.ShapeDtypeStruct((N, 2), jnp.float32),
    ],
)


def _tc2_body(p_ref, b1_ref, w2_ref, att2_ref, gp_ref, a2_ref):
    den = p_ref[0, 0][:, 32:33] + 1e-16
    out1 = jnp.concatenate(
        [p_ref[0, 0][:, :32] / den, p_ref[0, 1][:, :32] / den,
         p_ref[1, 0][:, :32] / den, p_ref[1, 1][:, :32] / den],
        axis=1)
    h = jnp.maximum(out1 + b1_ref[...], 0.0)
    g = jnp.dot(h, w2_ref[...], preferred_element_type=jnp.float32)
    gp_ref[...] = g
    a2_ref[...] = jnp.dot(g, att2_ref[...], preferred_element_type=jnp.float32)


_tc2 = pl.pallas_call(
    _tc2_body,
    grid=(GRID,),
    in_specs=[
        pl.BlockSpec((2, 2, ROW_BLK, 48), lambda i: (0, 0, i, 0)),
        pl.BlockSpec((1, H1), lambda i: (0, 0)),
        pl.BlockSpec((H1, H2), lambda i: (0, 0)),
        pl.BlockSpec((H2, 2), lambda i: (0, 0)),
    ],
    out_specs=[
        pl.BlockSpec((ROW_BLK, H2), lambda i: (i, 0)),
        pl.BlockSpec((ROW_BLK, 2), lambda i: (i, 0)),
    ],
    out_shape=[
        jax.ShapeDtypeStruct((N, H2), jnp.float32),
        jax.ShapeDtypeStruct((N, 2), jnp.float32),
    ],
)


def _tc3_body(p_ref, b2_ref, wc_ref, bc_ref, prob_ref, h2_ref):
    sacc = p_ref[0] + p_ref[1]
    h2 = sacc[:, :H2] / (sacc[:, 2:3] + 1e-16) + b2_ref[...]
    logits = jnp.dot(h2, wc_ref[...], preferred_element_type=jnp.float32)
    logits = logits + bc_ref[...]
    m = jnp.max(logits, axis=1, keepdims=True)
    ex = jnp.exp(logits - m)
    prob_ref[...] = ex / jnp.sum(ex, axis=1, keepdims=True)
    h2_ref[...] = h2


_tc3 = pl.pallas_call(
    _tc3_body,
    grid=(GRID,),
    in_specs=[
        pl.BlockSpec((2, ROW_BLK, 16), lambda i: (0, i, 0)),
        pl.BlockSpec((1, H2), lambda i: (0, 0)),
        pl.BlockSpec((H2, CLASSES), lambda i: (0, 0)),
        pl.BlockSpec((1, CLASSES), lambda i: (0, 0)),
    ],
    out_specs=[
        pl.BlockSpec((ROW_BLK, CLASSES), lambda i: (i, 0)),
        pl.BlockSpec((ROW_BLK, H2), lambda i: (i, 0)),
    ],
    out_shape=[
        jax.ShapeDtypeStruct((N, CLASSES), jnp.float32),
        jax.ShapeDtypeStruct((N, H2), jnp.float32),
    ],
)


def _make_agg1():
    F_q = 32
    ACC_W = 48
    EPT1 = E // NS
    CH1 = EPT1 // K
    mesh = plsc.VectorSubcoreMesh(core_axis_name="c", subcore_axis_name="s")

    def body(h_hbm, a_hbm, e_hbm, out_hbm,
             src_i, dst_i, wbuf, asrc, adst, gbuf, sbuf, zbuf, acc,
             gsem, ssem):
        c = lax.axis_index("c")
        s = lax.axis_index("s")

        pltpu.sync_copy(e_hbm.at[0, s], src_i)
        pltpu.sync_copy(e_hbm.at[1, s], dst_i)
        pltpu.sync_copy(a_hbm.at[0], asrc)
        pltpu.sync_copy(a_hbm.at[1], adst)

        lane = lax.iota(jnp.int32, 16)

        @plsc.parallel_loop(0, ZR)
        def zrow(i):
            for j in range(ACC_W // 16):
                zbuf[i, pl.ds(16 * j, 16)] = jnp.zeros((16,), jnp.float32)

        def zero_stripe():
            for r in range(STRIPE // ZR):
                pltpu.sync_copy(zbuf, acc.at[pl.ds(s * STRIPE + r * ZR, ZR)])

        zero_stripe()
        plsc.subcore_barrier()

        def wcompute(cn):
            @plsc.parallel_loop(0, K // 16)
            def wrow(j):
                sv = src_i[cn, 0, pl.ds(16 * j, 16)]
                dv = dst_i[cn, 0, pl.ds(16 * j, 16)]
                ev = plsc.load_gather(asrc, [sv]) + plsc.load_gather(adst, [dv])
                ev = jnp.where(ev > 0, ev, NEG_SLOPE * ev)
                wbuf[cn, 0, pl.ds(16 * j, 16)] = jnp.exp(ev)
                src_i[cn, 0, pl.ds(16 * j, 16)] = 4 * sv + c

        def scale_scatter(ci, b):
            @pl.when(ci >= 2)
            def _():
                pltpu.make_async_copy(
                    sbuf.at[b], acc.at[dst_i.at[ci, 0]], ssem.at[b]).wait()

            @plsc.parallel_loop(0, K // 16)
            def row16(jj):
                wv = wbuf[ci, 0, pl.ds(16 * jj, 16)]
                for l in range(16):
                    w_s = wv[l]
                    i = 16 * jj + l
                    for j in range(F_q // 16):
                        sbuf[b, i, pl.ds(16 * j, 16)] = (
                            gbuf[b, i, pl.ds(16 * j, 16)] * w_s)
                    sbuf[b, i, pl.ds(F_q, 16)] = jnp.where(
                        lane == 0, w_s, 0.0)
            pltpu.async_copy(
                sbuf.at[b], acc.at[dst_i.at[ci, 0]], ssem.at[b], add=True)

        def run_pass(prep):
            prep(0)
            pltpu.async_copy(h_hbm.at[src_i.at[0, 0]], gbuf.at[0], gsem.at[0])

            def pair(t, _):
                for b in range(2):
                    ci = 2 * t + b
                    if b == 0:
                        prep(ci + 1)
                        pltpu.async_copy(h_hbm.at[src_i.at[ci + 1, 0]],
                                         gbuf.at[1], gsem.at[1])
                    else:
                        @pl.when(t < CH1 // 2 - 1)
                        def _():
                            prep(ci + 1)
                            pltpu.async_copy(h_hbm.at[src_i.at[ci + 1, 0]],
                                             gbuf.at[0], gsem.at[0])
                    pltpu.make_async_copy(
                        h_hbm.at[src_i.at[ci, 0]], gbuf.at[b], gsem.at[b]).wait()
                    scale_scatter(ci, b)
                return 0
            lax.fori_loop(0, CH1 // 2, pair, 0)
            for b in range(2):
                pltpu.make_async_copy(
                    sbuf.at[b], acc.at[dst_i.at[0, 0]], ssem.at[b]).wait()
            plsc.subcore_barrier()
            for r in range(STRIPE // ZR):
                rows = pl.ds(s * STRIPE + r * ZR, ZR)
                pltpu.sync_copy(acc.at[rows], out_hbm.at[p_idx[0], c, rows])

        p_idx = [0]
        run_pass(wcompute)

        zero_stripe()

        def remap(cn):
            @plsc.parallel_loop(0, K // 16)
            def rrow(j):
                src_i[cn, 0, pl.ds(16 * j, 16)] = (
                    src_i[cn, 0, pl.ds(16 * j, 16)] + 2)
        plsc.subcore_barrier()

        p_idx[0] = 1
        run_pass(remap)

    return pl.kernel(
        body,
        out_type=jax.ShapeDtypeStruct((2, NC, N_ACC, ACC_W), jnp.float32),
        mesh=mesh,
        compiler_params=pltpu.CompilerParams(
            needs_layout_passes=False, use_tc_tiling_on_sc=False),
        scratch_types=[
            pltpu.VMEM((CH1, 1, K), jnp.int32),
            pltpu.VMEM((CH1, 1, K), jnp.int32),
            pltpu.VMEM((CH1, 1, K), jnp.float32),
            pltpu.VMEM((N,), jnp.float32),
            pltpu.VMEM((N,), jnp.float32),
            pltpu.VMEM((2, K, F_q), jnp.float32),
            pltpu.VMEM((2, K, ACC_W), jnp.float32),
            pltpu.VMEM((ZR, ACC_W), jnp.float32),
            pltpu.VMEM_SHARED((N_ACC, ACC_W), jnp.float32),
            pltpu.SemaphoreType.DMA((2,)),
            pltpu.SemaphoreType.DMA((2,)),
        ],
    )


def _make_agg2():
    ACC_W = 16
    mesh = plsc.VectorSubcoreMesh(core_axis_name="c", subcore_axis_name="s")

    def body(g_hbm, a_hbm, e_hbm, out_hbm,
             src_i, dst_i, g0, g1, asrc, adst, sbuf0, sbuf1, zbuf, acc, ssem):
        c = lax.axis_index("c")
        s = lax.axis_index("s")
        wid = s * NC + c

        pltpu.sync_copy(e_hbm.at[0, wid], src_i)
        pltpu.sync_copy(e_hbm.at[1, wid], dst_i)
        pltpu.sync_copy(a_hbm.at[0], asrc)
        pltpu.sync_copy(a_hbm.at[1], adst)
        pltpu.sync_copy(g_hbm.at[0], g0)
        pltpu.sync_copy(g_hbm.at[1], g1)

        lane = lax.iota(jnp.int32, 16)
        sbufs = (sbuf0, sbuf1)

        @plsc.parallel_loop(0, ZR)
        def zrow(i):
            zbuf[i, pl.ds(0, 16)] = jnp.zeros((16,), jnp.float32)

        @plsc.parallel_loop(0, K)
        def srow(i):
            sbuf0[i, pl.ds(0, 16)] = jnp.zeros((16,), jnp.float32)
            sbuf1[i, pl.ds(0, 16)] = jnp.zeros((16,), jnp.float32)
        for r in range(STRIPE // ZR):
            pltpu.sync_copy(zbuf, acc.at[pl.ds(s * STRIPE + r * ZR, ZR)])
        plsc.subcore_barrier()

        def fill_scatter(ci, b, sync):
            @pl.when(ci >= 3)
            def _():
                pltpu.make_async_copy(
                    sbufs[b], acc.at[dst_i.at[ci, 0]], ssem.at[b]).wait()

            @plsc.parallel_loop(0, K // 16)
            def erow(j):
                sv = src_i[ci, 0, pl.ds(16 * j, 16)]
                dv = dst_i[ci, 0, pl.ds(16 * j, 16)]
                ev = plsc.load_gather(asrc, [sv]) + plsc.load_gather(adst, [dv])
                ev = jnp.where(ev > 0, ev, NEG_SLOPE * ev)
                wv = jnp.exp(ev)
                g0v = plsc.load_gather(g0, [sv]) * wv
                g1v = plsc.load_gather(g1, [sv]) * wv
                rows = 16 * j + lane
                plsc.store_scatter(sbufs[b], [rows, lane * 0], g0v)
                plsc.store_scatter(sbufs[b], [rows, lane * 0 + 1], g1v)
                plsc.store_scatter(sbufs[b], [rows, lane * 0 + 2], wv)

            if sync:
                pltpu.sync_copy(sbufs[b], acc.at[dst_i.at[ci, 0]], add=True)
            else:
                pltpu.async_copy(
                    sbufs[b], acc.at[dst_i.at[ci, 0]], ssem.at[b], add=True)

        fill_scatter(0, 0, True)

        def pair(t, _):
            for b in range(2):
                ci = 1 + 2 * t + b
                fill_scatter(ci, b, False)
            return 0
        lax.fori_loop(0, (CH - 1) // 2, pair, 0)
        for b in range(2):
            pltpu.make_async_copy(
                sbufs[b], acc.at[dst_i.at[0, 0]], ssem.at[b]).wait()

        plsc.subcore_barrier()
        for r in range(STRIPE // ZR):
            rows = pl.ds(s * STRIPE + r * ZR, ZR)
            pltpu.sync_copy(acc.at[rows], out_hbm.at[c, rows])

    return pl.kernel(
        body,
        out_type=jax.ShapeDtypeStruct((NC, N_ACC, ACC_W), jnp.float32),
        mesh=mesh,
        compiler_params=pltpu.CompilerParams(
            needs_layout_passes=False, use_tc_tiling_on_sc=False),
        scratch_types=[
            pltpu.VMEM((CH, 1, K), jnp.int32),
            pltpu.VMEM((CH, 1, K), jnp.int32),
            pltpu.VMEM((N,), jnp.float32),
            pltpu.VMEM((N,), jnp.float32),
            pltpu.VMEM((N,), jnp.float32),
            pltpu.VMEM((N,), jnp.float32),
            pltpu.VMEM((K, ACC_W), jnp.float32),
            pltpu.VMEM((K, ACC_W), jnp.float32),
            pltpu.VMEM((ZR, ACC_W), jnp.float32),
            pltpu.VMEM_SHARED((N_ACC, ACC_W), jnp.float32),
            pltpu.SemaphoreType.DMA((2,)),
        ],
    )


_agg1 = _make_agg1()
_agg2 = _make_agg2()


def kernel(x, edge_index, W1, att_src1, att_dst1, b1,
           W2, att_src2, att_dst2, b2, Wc, bc):
    att1 = jnp.stack([att_src1, att_dst1], axis=1)
    att2 = jnp.stack([att_src2, att_dst2], axis=1)
    e32 = edge_index.reshape(2, NW, CH, 1, K)
    e16 = edge_index.reshape(2, NS, (E // NS) // K, 1, K)

    h1, a1 = _tc1(x, W1, att1)
    h4 = h1.reshape(4 * N, 32)
    a1t = a1.T
    parts1 = _agg1(h4, a1t, e16)
    g, a2 = _tc2(parts1, b1.reshape(1, H1), W2, att2)
    parts2 = _agg2(g.T, a2.T, e32)
    prob, h2 = _tc3(parts2, b2.reshape(1, H2), Wc, bc.reshape(1, CLASSES))
    return (prob, h2)


# speedup vs baseline: 1.0552x; 1.0552x over previous
import functools

import jax
import jax.numpy as jnp
from jax import lax
from jax.experimental import pallas as pl
from jax.experimental.pallas import tpu as pltpu
from jax.experimental.pallas import tpu_sc as plsc

N = 10000
E = 320000
F_IN = 128
H1 = 128
H2 = 2
CLASSES = 10
NEG_SLOPE = 0.2

NC = 2
NS = 16
NW = NC * NS
EPT = E // NW
K = 80
CH = EPT // K
N_ACC = 10240
STRIPE = N_ACC // NS
ZR = 128
ROW_BLK = 2000
GRID = N // ROW_BLK


def _tc1_body(x_ref, w_ref, att_ref, h_ref, a_ref):
    h = jnp.dot(x_ref[...], w_ref[...], preferred_element_type=jnp.float32)
    h_ref[...] = h
    a_ref[...] = jnp.dot(h, att_ref[...], preferred_element_type=jnp.float32)


_tc1 = pl.pallas_call(
    _tc1_body,
    grid=(GRID,),
    in_specs=[
        pl.BlockSpec((ROW_BLK, F_IN), lambda i: (i, 0)),
        pl.BlockSpec((F_IN, H1), lambda i: (0, 0)),
        pl.BlockSpec((H1, 2), lambda i: (0, 0)),
    ],
    out_specs=[
        pl.BlockSpec((ROW_BLK, H1), lambda i: (i, 0)),
        pl.BlockSpec((ROW_BLK, 2), lambda i: (i, 0)),
    ],
    out_shape=[
        jax.ShapeDtypeStruct((N, H1), jnp.float32),
        jax.ShapeDtypeStruct((N, 2), jnp.float32),
    ],
)


def _tc2_body(pa_ref, pb_ref, b1_ref, w2_ref, att2_ref, gp_ref, a2_ref):
    den = pa_ref[0][:, 32:33] + 1e-16
    out1 = jnp.concatenate(
        [pa_ref[0][:, :32] / den, pa_ref[1][:, :32] / den,
         pb_ref[0][:, :32] / den, pb_ref[1][:, :32] / den],
        axis=1)
    h = jnp.maximum(out1 + b1_ref[...], 0.0)
    g = jnp.dot(h, w2_ref[...], preferred_element_type=jnp.float32)
    gp_ref[...] = g
    a2_ref[...] = jnp.dot(g, att2_ref[...], preferred_element_type=jnp.float32)


_tc2 = pl.pallas_call(
    _tc2_body,
    grid=(GRID,),
    in_specs=[
        pl.BlockSpec((2, ROW_BLK, 48), lambda i: (0, i, 0)),
        pl.BlockSpec((2, ROW_BLK, 32), lambda i: (0, i, 0)),
        pl.BlockSpec((1, H1), lambda i: (0, 0)),
        pl.BlockSpec((H1, H2), lambda i: (0, 0)),
        pl.BlockSpec((H2, 2), lambda i: (0, 0)),
    ],
    out_specs=[
        pl.BlockSpec((ROW_BLK, H2), lambda i: (i, 0)),
        pl.BlockSpec((ROW_BLK, 2), lambda i: (i, 0)),
    ],
    out_shape=[
        jax.ShapeDtypeStruct((N, H2), jnp.float32),
        jax.ShapeDtypeStruct((N, 2), jnp.float32),
    ],
)


def _tc3_body(p_ref, b2_ref, wc_ref, bc_ref, prob_ref, h2_ref):
    sacc = p_ref[0] + p_ref[1]
    h2 = sacc[:, :H2] / (sacc[:, 2:3] + 1e-16) + b2_ref[...]
    logits = jnp.dot(h2, wc_ref[...], preferred_element_type=jnp.float32)
    logits = logits + bc_ref[...]
    m = jnp.max(logits, axis=1, keepdims=True)
    ex = jnp.exp(logits - m)
    prob_ref[...] = ex / jnp.sum(ex, axis=1, keepdims=True)
    h2_ref[...] = h2


_tc3 = pl.pallas_call(
    _tc3_body,
    grid=(GRID,),
    in_specs=[
        pl.BlockSpec((2, ROW_BLK, 16), lambda i: (0, i, 0)),
        pl.BlockSpec((1, H2), lambda i: (0, 0)),
        pl.BlockSpec((H2, CLASSES), lambda i: (0, 0)),
        pl.BlockSpec((1, CLASSES), lambda i: (0, 0)),
    ],
    out_specs=[
        pl.BlockSpec((ROW_BLK, CLASSES), lambda i: (i, 0)),
        pl.BlockSpec((ROW_BLK, H2), lambda i: (i, 0)),
    ],
    out_shape=[
        jax.ShapeDtypeStruct((N, CLASSES), jnp.float32),
        jax.ShapeDtypeStruct((N, H2), jnp.float32),
    ],
)


def _make_agg1(call_idx, with_denom):
    F_q = 32
    ACC_W = 48 if with_denom else 32
    EPT1 = E // NS
    CH1 = EPT1 // K
    mesh = plsc.VectorSubcoreMesh(core_axis_name="c", subcore_axis_name="s")

    def body(h_hbm, a_hbm, e_hbm, out_hbm,
             src_i, dst_i, wbuf, asrc, adst, gbuf, sbuf, zbuf, acc,
             gsem, ssem):
        c = lax.axis_index("c")
        s = lax.axis_index("s")

        pltpu.sync_copy(e_hbm.at[0, s], src_i)
        pltpu.sync_copy(e_hbm.at[1, s], dst_i)
        pltpu.sync_copy(a_hbm.at[0], asrc)
        pltpu.sync_copy(a_hbm.at[1], adst)

        @plsc.parallel_loop(0, ZR)
        def zrow(i):
            for j in range(ACC_W // 16):
                zbuf[i, pl.ds(16 * j, 16)] = jnp.zeros((16,), jnp.float32)
        for r in range(STRIPE // ZR):
            pltpu.sync_copy(zbuf, acc.at[pl.ds(s * STRIPE + r * ZR, ZR)])
        plsc.subcore_barrier()

        lane = lax.iota(jnp.int32, 16)

        def wcompute(cn):
            @plsc.parallel_loop(0, K // 16)
            def wrow(j):
                sv = src_i[cn, 0, pl.ds(16 * j, 16)]
                dv = dst_i[cn, 0, pl.ds(16 * j, 16)]
                ev = plsc.load_gather(asrc, [sv]) + plsc.load_gather(adst, [dv])
                ev = jnp.where(ev > 0, ev, NEG_SLOPE * ev)
                wbuf[cn, 0, pl.ds(16 * j, 16)] = jnp.exp(ev)
                src_i[cn, 0, pl.ds(16 * j, 16)] = 4 * sv + (2 * call_idx + c)

        def scale_scatter(ci, b):
            @pl.when(ci >= 2)
            def _():
                pltpu.make_async_copy(
                    sbuf.at[b], acc.at[dst_i.at[ci, 0]], ssem.at[b]).wait()

            @plsc.parallel_loop(0, K // 16)
            def row16(jj):
                wv = wbuf[ci, 0, pl.ds(16 * jj, 16)]
                for l in range(16):
                    w_s = wv[l]
                    i = 16 * jj + l
                    for j in range(F_q // 16):
                        sbuf[b, i, pl.ds(16 * j, 16)] = (
                            gbuf[b, i, pl.ds(16 * j, 16)] * w_s)
                    if with_denom:
                        sbuf[b, i, pl.ds(F_q, 16)] = jnp.where(
                            lane == 0, w_s, 0.0)
            pltpu.async_copy(
                sbuf.at[b], acc.at[dst_i.at[ci, 0]], ssem.at[b], add=True)

        wcompute(0)
        pltpu.async_copy(h_hbm.at[src_i.at[0, 0]], gbuf.at[0], gsem.at[0])

        def pair(t, _):
            for b in range(2):
                ci = 2 * t + b
                if b == 0:
                    wcompute(ci + 1)
                    pltpu.async_copy(
                        h_hbm.at[src_i.at[ci + 1, 0]], gbuf.at[1], gsem.at[1])
                else:
                    @pl.when(t < CH1 // 2 - 1)
                    def _():
                        wcompute(ci + 1)
                        pltpu.async_copy(
                            h_hbm.at[src_i.at[ci + 1, 0]], gbuf.at[0], gsem.at[0])
                pltpu.make_async_copy(
                    h_hbm.at[src_i.at[ci, 0]], gbuf.at[b], gsem.at[b]).wait()
                scale_scatter(ci, b)
            return 0
        lax.fori_loop(0, CH1 // 2, pair, 0)
        for b in range(2):
            pltpu.make_async_copy(
                sbuf.at[b], acc.at[dst_i.at[0, 0]], ssem.at[b]).wait()

        plsc.subcore_barrier()
        for r in range(STRIPE // ZR):
            rows = pl.ds(s * STRIPE + r * ZR, ZR)
            pltpu.sync_copy(acc.at[rows], out_hbm.at[c, rows])

    return pl.kernel(
        body,
        out_type=jax.ShapeDtypeStruct((NC, N_ACC, ACC_W), jnp.float32),
        mesh=mesh,
        compiler_params=pltpu.CompilerParams(
            needs_layout_passes=False, use_tc_tiling_on_sc=False),
        scratch_types=[
            pltpu.VMEM((CH1, 1, K), jnp.int32),
            pltpu.VMEM((CH1, 1, K), jnp.int32),
            pltpu.VMEM((CH1, 1, K), jnp.float32),
            pltpu.VMEM((N,), jnp.float32),
            pltpu.VMEM((N,), jnp.float32),
            pltpu.VMEM((2, K, F_q), jnp.float32),
            pltpu.VMEM((2, K, ACC_W), jnp.float32),
            pltpu.VMEM((ZR, ACC_W), jnp.float32),
            pltpu.VMEM_SHARED((N_ACC, ACC_W), jnp.float32),
            pltpu.SemaphoreType.DMA((2,)),
            pltpu.SemaphoreType.DMA((2,)),
        ],
    )


def _make_agg2():
    ACC_W = 16
    mesh = plsc.VectorSubcoreMesh(core_axis_name="c", subcore_axis_name="s")

    def body(g_hbm, a_hbm, e_hbm, out_hbm,
             src_i, dst_i, g0, g1, asrc, adst, sbuf0, sbuf1, zbuf, acc, ssem):
        c = lax.axis_index("c")
        s = lax.axis_index("s")
        wid = s * NC + c

        pltpu.sync_copy(e_hbm.at[0, wid], src_i)
        pltpu.sync_copy(e_hbm.at[1, wid], dst_i)
        pltpu.sync_copy(a_hbm.at[0], asrc)
        pltpu.sync_copy(a_hbm.at[1], adst)
        pltpu.sync_copy(g_hbm.at[0], g0)
        pltpu.sync_copy(g_hbm.at[1], g1)

        lane = lax.iota(jnp.int32, 16)
        sbufs = (sbuf0, sbuf1)

        @plsc.parallel_loop(0, ZR)
        def zrow(i):
            zbuf[i, pl.ds(0, 16)] = jnp.zeros((16,), jnp.float32)

        @plsc.parallel_loop(0, K)
        def srow(i):
            sbuf0[i, pl.ds(0, 16)] = jnp.zeros((16,), jnp.float32)
            sbuf1[i, pl.ds(0, 16)] = jnp.zeros((16,), jnp.float32)
        for r in range(STRIPE // ZR):
            pltpu.sync_copy(zbuf, acc.at[pl.ds(s * STRIPE + r * ZR, ZR)])
        plsc.subcore_barrier()

        def fill_scatter(ci, b, sync):
            @pl.when(ci >= 3)
            def _():
                pltpu.make_async_copy(
                    sbufs[b], acc.at[dst_i.at[ci, 0]], ssem.at[b]).wait()

            @plsc.parallel_loop(0, K // 16)
            def erow(j):
                sv = src_i[ci, 0, pl.ds(16 * j, 16)]
                dv = dst_i[ci, 0, pl.ds(16 * j, 16)]
                ev = plsc.load_gather(asrc, [sv]) + plsc.load_gather(adst, [dv])
                ev = jnp.where(ev > 0, ev, NEG_SLOPE * ev)
                wv = jnp.exp(ev)
                g0v = plsc.load_gather(g0, [sv]) * wv
                g1v = plsc.load_gather(g1, [sv]) * wv
                rows = 16 * j + lane
                plsc.store_scatter(sbufs[b], [rows, lane * 0], g0v)
                plsc.store_scatter(sbufs[b], [rows, lane * 0 + 1], g1v)
                plsc.store_scatter(sbufs[b], [rows, lane * 0 + 2], wv)

            if sync:
                pltpu.sync_copy(sbufs[b], acc.at[dst_i.at[ci, 0]], add=True)
            else:
                pltpu.async_copy(
                    sbufs[b], acc.at[dst_i.at[ci, 0]], ssem.at[b], add=True)

        fill_scatter(0, 0, True)

        def pair(t, _):
            for b in range(2):
                ci = 1 + 2 * t + b
                fill_scatter(ci, b, False)
            return 0
        lax.fori_loop(0, (CH - 1) // 2, pair, 0)
        for b in range(2):
            pltpu.make_async_copy(
                sbufs[b], acc.at[dst_i.at[0, 0]], ssem.at[b]).wait()

        plsc.subcore_barrier()
        for r in range(STRIPE // ZR):
            rows = pl.ds(s * STRIPE + r * ZR, ZR)
            pltpu.sync_copy(acc.at[rows], out_hbm.at[c, rows])

    return pl.kernel(
        body,
        out_type=jax.ShapeDtypeStruct((NC, N_ACC, ACC_W), jnp.float32),
        mesh=mesh,
        compiler_params=pltpu.CompilerParams(
            needs_layout_passes=False, use_tc_tiling_on_sc=False),
        scratch_types=[
            pltpu.VMEM((CH, 1, K), jnp.int32),
            pltpu.VMEM((CH, 1, K), jnp.int32),
            pltpu.VMEM((N,), jnp.float32),
            pltpu.VMEM((N,), jnp.float32),
            pltpu.VMEM((N,), jnp.float32),
            pltpu.VMEM((N,), jnp.float32),
            pltpu.VMEM((K, ACC_W), jnp.float32),
            pltpu.VMEM((K, ACC_W), jnp.float32),
            pltpu.VMEM((ZR, ACC_W), jnp.float32),
            pltpu.VMEM_SHARED((N_ACC, ACC_W), jnp.float32),
            pltpu.SemaphoreType.DMA((2,)),
        ],
    )


_agg1a = _make_agg1(0, True)
_agg1b = _make_agg1(1, False)
_agg2 = _make_agg2()


def kernel(x, edge_index, W1, att_src1, att_dst1, b1,
           W2, att_src2, att_dst2, b2, Wc, bc):
    att1 = jnp.stack([att_src1, att_dst1], axis=1)
    att2 = jnp.stack([att_src2, att_dst2], axis=1)
    e32 = edge_index.reshape(2, NW, CH, 1, K)
    e16 = edge_index.reshape(2, NS, (E // NS) // K, 1, K)

    h1, a1 = _tc1(x, W1, att1)
    h4 = h1.reshape(4 * N, 32)
    a1t = a1.T
    pa = _agg1a(h4, a1t, e16)
    pb = _agg1b(h4, a1t, e16)
    g, a2 = _tc2(pa, pb, b1.reshape(1, H1), W2, att2)
    parts2 = _agg2(g.T, a2.T, e32)
    prob, h2 = _tc3(parts2, b2.reshape(1, H2), Wc, bc.reshape(1, CLASSES))
    return (prob, h2)
